# baseline (speedup 1.0000x reference)
import functools

import jax
import jax.numpy as jnp
from jax import lax
from jax.experimental import pallas as pl
from jax.experimental.pallas import tpu as pltpu
from jax.experimental.pallas import tpu_sc as plsc

NC = 2
NS = 16
CH = 128


IB = 16
CORE_FRAC16 = 5


def _sc_scatter(nfeat, row2d, col2d, z_rst, z_deg, ones_blk, *, n_acc,
                cpt0, cpt1):
  d = nfeat.shape[1]
  init_rows = n_acc // NS
  nb0, nb1 = cpt0 // IB, cpt1 // IB

  mesh = plsc.VectorSubcoreMesh(core_axis_name="c", subcore_axis_name="s",
                                num_cores=NC, num_subcores=NS)

  @functools.partial(
      pl.kernel,
      out_type=(
          jax.ShapeDtypeStruct((NC, n_acc, d), jnp.float32),
          jax.ShapeDtypeStruct((NC * n_acc,), jnp.float32),
      ),
      mesh=mesh,
      scratch_types=[
          pltpu.VMEM_SHARED((n_acc, d), jnp.float32),
          pltpu.VMEM_SHARED((n_acc,), jnp.float32),
          pltpu.VMEM((IB, CH), jnp.int32),
          pltpu.VMEM((IB, CH), jnp.int32),
          pltpu.VMEM((IB, CH), jnp.int32),
          pltpu.VMEM((IB, CH), jnp.int32),
          pltpu.VMEM((CH,), jnp.float32),
          pltpu.VMEM((init_rows,), jnp.float32),
          pltpu.VMEM((CH, d), jnp.float32),
          pltpu.VMEM((CH, d), jnp.float32),
          pltpu.SemaphoreType.DMA,
          pltpu.SemaphoreType.DMA,
          pltpu.SemaphoreType.DMA,
          pltpu.SemaphoreType.DMA,
      ],
  )
  def k(nfeat_hbm, row_hbm, col_hbm, zrst_hbm, zdeg_hbm, ones_hbm,
        rst_out, deg_out, sh_rst, sh_deg, row_a, col_a, row_b, col_b,
        ones_v, tmp_v, gbuf_a, gbuf_b, gsem_a, gsem_b, isem_a, isem_b):
    c = lax.axis_index("c")
    s = lax.axis_index("s")
    tile = c * NS + s

    pltpu.sync_copy(zrst_hbm.at[pl.ds(s * init_rows, init_rows)],
                    sh_rst.at[pl.ds(s * init_rows, init_rows)])
    pltpu.sync_copy(zdeg_hbm.at[pl.ds(s * init_rows, init_rows)], tmp_v)
    pltpu.sync_copy(tmp_v, sh_deg.at[pl.ds(s * init_rows, init_rows)])
    pltpu.sync_copy(ones_hbm, ones_v)
    base = jnp.where(c == 0, s * cpt0, NS * cpt0 + s * cpt1)
    nb = jnp.where(c == 0, nb0, nb1)
    plsc.subcore_barrier()

    gbufs = (gbuf_a, gbuf_b)
    gsems = (gsem_a, gsem_b)

    def load_idx(blk, rowb, colb, isem):
      pltpu.async_copy(row_hbm.at[pl.ds(base + blk * IB, IB)], rowb, isem)
      pltpu.async_copy(col_hbm.at[pl.ds(base + blk * IB, IB)], colb, isem)

    def wait_idx(blk, rowb, colb, isem):
      pltpu.make_async_copy(row_hbm.at[pl.ds(base + blk * IB, IB)],
                            rowb, isem).wait()
      pltpu.make_async_copy(col_hbm.at[pl.ds(base + blk * IB, IB)],
                            colb, isem).wait()

    load_idx(0, row_a, col_a, isem_a)

    def blk_body(k, _):
      def run(rowb, colb, isem, n_rowb, n_colb, n_isem):
        wait_idx(k, rowb, colb, isem)

        @pl.when(k + 1 < nb)
        def _():
          load_idx(k + 1, n_rowb, n_colb, n_isem)

        def gissue(off, p):
          for h in range(2):
            pltpu.async_copy(nfeat_hbm.at[colb.at[off, pl.ds(h * 64, 64)]],
                             gbufs[p].at[pl.ds(h * 64, 64)], gsems[p])

        gissue(0, 0)
        gissue(1, 1)
        for off in range(IB):
          p = off % 2
          pltpu.make_async_copy(nfeat_hbm.at[colb.at[off]],
                                gbufs[p], gsems[p]).wait()
          pltpu.sync_copy(gbufs[p], sh_rst.at[rowb.at[off]], add=True)
          pltpu.sync_copy(ones_v, sh_deg.at[rowb.at[off]], add=True)
          if off + 2 < IB:
            gissue(off + 2, p)

      @pl.when(k % 2 == 0)
      def _():
        run(row_a, col_a, isem_a, row_b, col_b, isem_b)

      @pl.when(k % 2 == 1)
      def _():
        run(row_b, col_b, isem_b, row_a, col_a, isem_a)

      return 0

    lax.fori_loop(0, nb, blk_body, 0)
    plsc.subcore_barrier()

    pltpu.sync_copy(sh_rst.at[pl.ds(s * init_rows, init_rows)],
                    rst_out.at[c, pl.ds(s * init_rows, init_rows)])
    pltpu.sync_copy(sh_deg.at[pl.ds(s * init_rows, init_rows)], tmp_v)
    pltpu.sync_copy(tmp_v, deg_out.at[pl.ds(c * n_acc + s * init_rows,
                                            init_rows)])

  return k(nfeat, row2d, col2d, z_rst, z_deg, ones_blk)


def _tc_finish_body(rp_ref, dp_ref, nf_ref, w_ref, b_ref, out_ref):
  acc = rp_ref[0] + rp_ref[1] + nf_ref[...]
  deg = dp_ref[:, 0:1] + dp_ref[:, 1:2] + 1.0
  rst = acc / deg
  out_ref[...] = lax.dot_general(
      rst, w_ref[...], (((1,), (1,)), ((), ())),
      preferred_element_type=jnp.float32) + b_ref[...]


def _tc_finish(rst_part, deg_part, nfeat, W, b2, *, blk):
  n, d = nfeat.shape
  grid = n // blk
  return pl.pallas_call(
      _tc_finish_body,
      grid=(grid,),
      in_specs=[
          pl.BlockSpec((NC, blk, d), lambda i: (0, i, 0)),
          pl.BlockSpec((blk, NC), lambda i: (i, 0)),
          pl.BlockSpec((blk, d), lambda i: (i, 0)),
          pl.BlockSpec((d, d), lambda i: (0, 0)),
          pl.BlockSpec((1, d), lambda i: (0, 0)),
      ],
      out_specs=pl.BlockSpec((blk, d), lambda i: (i, 0)),
      out_shape=jax.ShapeDtypeStruct((n, d), jnp.float32),
  )(rst_part, deg_part, nfeat, W, b2)


def kernel(nfeat, efeat, edge_index, W, b):
  del efeat
  n, d = nfeat.shape
  e = edge_index.shape[1]

  chunks_per_tile = -(-(-(-e // (NC * NS * CH))) // IB) * IB
  cpt0 = (NC * chunks_per_tile * CORE_FRAC16 // 16) // IB * IB
  cpt1 = NC * chunks_per_tile - cpt0
  e_pad = NS * CH * (cpt0 + cpt1)
  n_acc = -(-(n + 1) // (NS * 8)) * NS * 8

  row = edge_index[0].astype(jnp.int32)
  col = edge_index[1].astype(jnp.int32)
  row = jnp.concatenate([row, jnp.full((e_pad - e,), n, jnp.int32)])
  col = jnp.concatenate([col, jnp.zeros((e_pad - e,), jnp.int32)])
  row = row.reshape(-1, CH)
  col = col.reshape(-1, CH)

  z_rst = jnp.zeros((n_acc, d), jnp.float32)
  z_deg = jnp.zeros((n_acc,), jnp.float32)
  ones_blk = jnp.ones((CH,), jnp.float32)

  rst_part, deg_part = _sc_scatter(
      nfeat, row, col, z_rst, z_deg, ones_blk,
      n_acc=n_acc, cpt0=cpt0, cpt1=cpt1)

  deg_t = deg_part.reshape(NC, n_acc).T
  return _tc_finish(rst_part, deg_t, nfeat, W, b.reshape(1, d), blk=1000)


# speedup vs baseline: 1.0492x; 1.0492x over previous
import functools

import jax
import jax.numpy as jnp
from jax import lax
from jax.experimental import pallas as pl
from jax.experimental.pallas import tpu as pltpu
from jax.experimental.pallas import tpu_sc as plsc

NC = 2
NS = 16
CH = 128


IB = 16
CORE_FRAC16 = 11


def _sc_scatter(nfeat, row2d, col2d, z_rst, z_deg, ones_blk, *, n_acc,
                cpt0, cpt1):
  d = nfeat.shape[1]
  init_rows = n_acc // NS
  nb0, nb1 = cpt0 // IB, cpt1 // IB

  mesh = plsc.VectorSubcoreMesh(core_axis_name="c", subcore_axis_name="s",
                                num_cores=NC, num_subcores=NS)

  @functools.partial(
      pl.kernel,
      out_type=(
          jax.ShapeDtypeStruct((NC, n_acc, d), jnp.float32),
          jax.ShapeDtypeStruct((NC * n_acc,), jnp.float32),
      ),
      mesh=mesh,
      scratch_types=[
          pltpu.VMEM_SHARED((n_acc, d), jnp.float32),
          pltpu.VMEM_SHARED((n_acc,), jnp.float32),
          pltpu.VMEM((IB, CH), jnp.int32),
          pltpu.VMEM((IB, CH), jnp.int32),
          pltpu.VMEM((IB, CH), jnp.int32),
          pltpu.VMEM((IB, CH), jnp.int32),
          pltpu.VMEM((CH,), jnp.float32),
          pltpu.VMEM((init_rows,), jnp.float32),
          pltpu.VMEM((CH, d), jnp.float32),
          pltpu.VMEM((CH, d), jnp.float32),
          pltpu.SemaphoreType.DMA,
          pltpu.SemaphoreType.DMA,
          pltpu.SemaphoreType.DMA,
          pltpu.SemaphoreType.DMA,
      ],
  )
  def k(nfeat_hbm, row_hbm, col_hbm, zrst_hbm, zdeg_hbm, ones_hbm,
        rst_out, deg_out, sh_rst, sh_deg, row_a, col_a, row_b, col_b,
        ones_v, tmp_v, gbuf_a, gbuf_b, gsem_a, gsem_b, isem_a, isem_b):
    c = lax.axis_index("c")
    s = lax.axis_index("s")
    tile = c * NS + s

    pltpu.sync_copy(zrst_hbm.at[pl.ds(s * init_rows, init_rows)],
                    sh_rst.at[pl.ds(s * init_rows, init_rows)])
    pltpu.sync_copy(zdeg_hbm.at[pl.ds(s * init_rows, init_rows)], tmp_v)
    pltpu.sync_copy(tmp_v, sh_deg.at[pl.ds(s * init_rows, init_rows)])
    pltpu.sync_copy(ones_hbm, ones_v)
    base = jnp.where(c == 0, s * cpt0, NS * cpt0 + s * cpt1)
    nb = jnp.where(c == 0, nb0, nb1)
    plsc.subcore_barrier()

    gbufs = (gbuf_a, gbuf_b)
    gsems = (gsem_a, gsem_b)

    def load_idx(blk, rowb, colb, isem):
      pltpu.async_copy(row_hbm.at[pl.ds(base + blk * IB, IB)], rowb, isem)
      pltpu.async_copy(col_hbm.at[pl.ds(base + blk * IB, IB)], colb, isem)

    def wait_idx(blk, rowb, colb, isem):
      pltpu.make_async_copy(row_hbm.at[pl.ds(base + blk * IB, IB)],
                            rowb, isem).wait()
      pltpu.make_async_copy(col_hbm.at[pl.ds(base + blk * IB, IB)],
                            colb, isem).wait()

    load_idx(0, row_a, col_a, isem_a)

    def blk_body(k, _):
      def run(rowb, colb, isem, n_rowb, n_colb, n_isem):
        wait_idx(k, rowb, colb, isem)

        @pl.when(k + 1 < nb)
        def _():
          load_idx(k + 1, n_rowb, n_colb, n_isem)

        def gissue(off, p):
          for h in range(2):
            pltpu.async_copy(nfeat_hbm.at[colb.at[off, pl.ds(h * 64, 64)]],
                             gbufs[p].at[pl.ds(h * 64, 64)], gsems[p])

        gissue(0, 0)
        gissue(1, 1)
        for off in range(IB):
          p = off % 2
          pltpu.make_async_copy(nfeat_hbm.at[colb.at[off]],
                                gbufs[p], gsems[p]).wait()
          pltpu.sync_copy(gbufs[p], sh_rst.at[rowb.at[off]], add=True)
          pltpu.sync_copy(ones_v, sh_deg.at[rowb.at[off]], add=True)
          if off + 2 < IB:
            gissue(off + 2, p)

      @pl.when(k % 2 == 0)
      def _():
        run(row_a, col_a, isem_a, row_b, col_b, isem_b)

      @pl.when(k % 2 == 1)
      def _():
        run(row_b, col_b, isem_b, row_a, col_a, isem_a)

      return 0

    lax.fori_loop(0, nb, blk_body, 0)
    plsc.subcore_barrier()

    pltpu.sync_copy(sh_rst.at[pl.ds(s * init_rows, init_rows)],
                    rst_out.at[c, pl.ds(s * init_rows, init_rows)])
    pltpu.sync_copy(sh_deg.at[pl.ds(s * init_rows, init_rows)], tmp_v)
    pltpu.sync_copy(tmp_v, deg_out.at[pl.ds(c * n_acc + s * init_rows,
                                            init_rows)])

  return k(nfeat, row2d, col2d, z_rst, z_deg, ones_blk)


def _tc_finish_body(rp_ref, dp_ref, nf_ref, w_ref, b_ref, out_ref):
  acc = rp_ref[0] + rp_ref[1] + nf_ref[...]
  deg = dp_ref[:, 0:1] + dp_ref[:, 1:2] + 1.0
  rst = acc / deg
  out_ref[...] = lax.dot_general(
      rst, w_ref[...], (((1,), (1,)), ((), ())),
      preferred_element_type=jnp.float32) + b_ref[...]


def _tc_finish(rst_part, deg_part, nfeat, W, b2, *, blk):
  n, d = nfeat.shape
  grid = n // blk
  return pl.pallas_call(
      _tc_finish_body,
      grid=(grid,),
      in_specs=[
          pl.BlockSpec((NC, blk, d), lambda i: (0, i, 0)),
          pl.BlockSpec((blk, NC), lambda i: (i, 0)),
          pl.BlockSpec((blk, d), lambda i: (i, 0)),
          pl.BlockSpec((d, d), lambda i: (0, 0)),
          pl.BlockSpec((1, d), lambda i: (0, 0)),
      ],
      out_specs=pl.BlockSpec((blk, d), lambda i: (i, 0)),
      out_shape=jax.ShapeDtypeStruct((n, d), jnp.float32),
  )(rst_part, deg_part, nfeat, W, b2)


def kernel(nfeat, efeat, edge_index, W, b):
  del efeat
  n, d = nfeat.shape
  e = edge_index.shape[1]

  chunks_per_tile = -(-(-(-e // (NC * NS * CH))) // IB) * IB
  cpt0 = (NC * chunks_per_tile * CORE_FRAC16 // 16) // IB * IB
  cpt1 = NC * chunks_per_tile - cpt0
  e_pad = NS * CH * (cpt0 + cpt1)
  n_acc = -(-(n + 1) // (NS * 8)) * NS * 8

  row = edge_index[0].astype(jnp.int32)
  col = edge_index[1].astype(jnp.int32)
  row = jnp.concatenate([row, jnp.full((e_pad - e,), n, jnp.int32)])
  col = jnp.concatenate([col, jnp.zeros((e_pad - e,), jnp.int32)])
  row = row.reshape(-1, CH)
  col = col.reshape(-1, CH)

  z_rst = jnp.zeros((n_acc, d), jnp.float32)
  z_deg = jnp.zeros((n_acc,), jnp.float32)
  ones_blk = jnp.ones((CH,), jnp.float32)

  rst_part, deg_part = _sc_scatter(
      nfeat, row, col, z_rst, z_deg, ones_blk,
      n_acc=n_acc, cpt0=cpt0, cpt1=cpt1)

  deg_t = deg_part.reshape(NC, n_acc).T
  return _tc_finish(rst_part, deg_t, nfeat, W, b.reshape(1, d), blk=1000)


# speedup vs baseline: 1.0525x; 1.0031x over previous
import functools

import jax
import jax.numpy as jnp
from jax import lax
from jax.experimental import pallas as pl
from jax.experimental.pallas import tpu as pltpu
from jax.experimental.pallas import tpu_sc as plsc

NC = 2
NS = 16
CH = 128


IB = 16
CORE_FRAC16 = 11


def _sc_scatter(nfeat, row2d, col2d, z_rst, z_deg, ones_blk, *, n_acc,
                cpt0, cpt1):
  d = nfeat.shape[1]
  init_rows = n_acc // NS
  nb0, nb1 = cpt0 // IB, cpt1 // IB

  mesh = plsc.VectorSubcoreMesh(core_axis_name="c", subcore_axis_name="s",
                                num_cores=NC, num_subcores=NS)

  @functools.partial(
      pl.kernel,
      out_type=(
          jax.ShapeDtypeStruct((NC, n_acc, d), jnp.float32),
          jax.ShapeDtypeStruct((NC * n_acc,), jnp.float32),
      ),
      mesh=mesh,
      scratch_types=[
          pltpu.VMEM_SHARED((n_acc, d), jnp.float32),
          pltpu.VMEM_SHARED((n_acc,), jnp.float32),
          pltpu.VMEM((IB, CH), jnp.int32),
          pltpu.VMEM((IB, CH), jnp.int32),
          pltpu.VMEM((IB, CH), jnp.int32),
          pltpu.VMEM((IB, CH), jnp.int32),
          pltpu.VMEM((CH,), jnp.float32),
          pltpu.VMEM((init_rows,), jnp.float32),
          pltpu.VMEM((CH, d), jnp.float32),
          pltpu.VMEM((CH, d), jnp.float32),
          pltpu.SemaphoreType.DMA,
          pltpu.SemaphoreType.DMA,
          pltpu.SemaphoreType.DMA,
          pltpu.SemaphoreType.DMA,
      ],
  )
  def k(nfeat_hbm, row_hbm, col_hbm, zrst_hbm, zdeg_hbm, ones_hbm,
        rst_out, deg_out, sh_rst, sh_deg, row_a, col_a, row_b, col_b,
        ones_v, tmp_v, gbuf_a, gbuf_b, gsem_a, gsem_b, isem_a, isem_b):
    c = lax.axis_index("c")
    s = lax.axis_index("s")
    tile = c * NS + s

    pltpu.sync_copy(zrst_hbm,
                    sh_rst.at[pl.ds(s * init_rows, init_rows)])
    pltpu.sync_copy(zdeg_hbm, tmp_v)
    pltpu.sync_copy(tmp_v, sh_deg.at[pl.ds(s * init_rows, init_rows)])
    pltpu.sync_copy(ones_hbm, ones_v)
    base = jnp.where(c == 0, s * cpt0, NS * cpt0 + s * cpt1)
    nb = jnp.where(c == 0, nb0, nb1)
    plsc.subcore_barrier()

    gbufs = (gbuf_a, gbuf_b)
    gsems = (gsem_a, gsem_b)

    def load_idx(blk, rowb, colb, isem):
      pltpu.async_copy(row_hbm.at[pl.ds(base + blk * IB, IB)], rowb, isem)
      pltpu.async_copy(col_hbm.at[pl.ds(base + blk * IB, IB)], colb, isem)

    def wait_idx(blk, rowb, colb, isem):
      pltpu.make_async_copy(row_hbm.at[pl.ds(base + blk * IB, IB)],
                            rowb, isem).wait()
      pltpu.make_async_copy(col_hbm.at[pl.ds(base + blk * IB, IB)],
                            colb, isem).wait()

    load_idx(0, row_a, col_a, isem_a)

    def blk_body(k, _):
      def run(rowb, colb, isem, n_rowb, n_colb, n_isem):
        wait_idx(k, rowb, colb, isem)

        @pl.when(k + 1 < nb)
        def _():
          load_idx(k + 1, n_rowb, n_colb, n_isem)

        def gissue(off, p):
          for h in range(2):
            pltpu.async_copy(nfeat_hbm.at[colb.at[off, pl.ds(h * 64, 64)]],
                             gbufs[p].at[pl.ds(h * 64, 64)], gsems[p])

        gissue(0, 0)
        gissue(1, 1)
        for off in range(IB):
          p = off % 2
          pltpu.make_async_copy(nfeat_hbm.at[colb.at[off]],
                                gbufs[p], gsems[p]).wait()
          pltpu.sync_copy(gbufs[p], sh_rst.at[rowb.at[off]], add=True)
          pltpu.sync_copy(ones_v, sh_deg.at[rowb.at[off]], add=True)
          if off + 2 < IB:
            gissue(off + 2, p)

      @pl.when(k % 2 == 0)
      def _():
        run(row_a, col_a, isem_a, row_b, col_b, isem_b)

      @pl.when(k % 2 == 1)
      def _():
        run(row_b, col_b, isem_b, row_a, col_a, isem_a)

      return 0

    lax.fori_loop(0, nb, blk_body, 0)
    plsc.subcore_barrier()

    pltpu.sync_copy(sh_rst.at[pl.ds(s * init_rows, init_rows)],
                    rst_out.at[c, pl.ds(s * init_rows, init_rows)])
    pltpu.sync_copy(sh_deg.at[pl.ds(s * init_rows, init_rows)], tmp_v)
    pltpu.sync_copy(tmp_v, deg_out.at[pl.ds(c * n_acc + s * init_rows,
                                            init_rows)])

  return k(nfeat, row2d, col2d, z_rst, z_deg, ones_blk)


def _tc_finish_body(rp_ref, dp_ref, nf_ref, w_ref, b_ref, out_ref):
  acc = rp_ref[0] + rp_ref[1] + nf_ref[...]
  deg = dp_ref[:, 0:1] + dp_ref[:, 1:2] + 1.0
  rst = acc / deg
  out_ref[...] = lax.dot_general(
      rst, w_ref[...], (((1,), (1,)), ((), ())),
      preferred_element_type=jnp.float32) + b_ref[...]


def _tc_finish(rst_part, deg_part, nfeat, W, b2, *, blk):
  n, d = nfeat.shape
  grid = n // blk
  return pl.pallas_call(
      _tc_finish_body,
      grid=(grid,),
      in_specs=[
          pl.BlockSpec((NC, blk, d), lambda i: (0, i, 0)),
          pl.BlockSpec((blk, NC), lambda i: (i, 0)),
          pl.BlockSpec((blk, d), lambda i: (i, 0)),
          pl.BlockSpec((d, d), lambda i: (0, 0)),
          pl.BlockSpec((1, d), lambda i: (0, 0)),
      ],
      out_specs=pl.BlockSpec((blk, d), lambda i: (i, 0)),
      out_shape=jax.ShapeDtypeStruct((n, d), jnp.float32),
  )(rst_part, deg_part, nfeat, W, b2)


def kernel(nfeat, efeat, edge_index, W, b):
  del efeat
  n, d = nfeat.shape
  e = edge_index.shape[1]

  chunks_per_tile = -(-(-(-e // (NC * NS * CH))) // IB) * IB
  cpt0 = (NC * chunks_per_tile * CORE_FRAC16 // 16) // IB * IB
  cpt1 = NC * chunks_per_tile - cpt0
  e_pad = NS * CH * (cpt0 + cpt1)
  n_acc = -(-(n + 1) // (NS * 8)) * NS * 8

  row = edge_index[0].astype(jnp.int32)
  col = edge_index[1].astype(jnp.int32)
  row = jnp.concatenate([row, jnp.full((e_pad - e,), n, jnp.int32)])
  col = jnp.concatenate([col, jnp.zeros((e_pad - e,), jnp.int32)])
  row = row.reshape(-1, CH)
  col = col.reshape(-1, CH)

  z_rst = jnp.zeros((n_acc // NS, d), jnp.float32)
  z_deg = jnp.zeros((n_acc // NS,), jnp.float32)
  ones_blk = jnp.ones((CH,), jnp.float32)

  rst_part, deg_part = _sc_scatter(
      nfeat, row, col, z_rst, z_deg, ones_blk,
      n_acc=n_acc, cpt0=cpt0, cpt1=cpt1)

  deg_t = deg_part.reshape(NC, n_acc).T
  return _tc_finish(rst_part, deg_t, nfeat, W, b.reshape(1, d), blk=1000)
